# baseline (speedup 1.0000x reference)
import functools

import jax
import jax.numpy as jnp
from jax import lax
from jax.experimental import pallas as pl
from jax.experimental.pallas import tpu as pltpu
from jax.experimental.pallas import tpu_sc as plsc

L = 16
LPA = 128
COUNT = 6
BUCKET_SIZE = 5
ACTIVE = 32
NW = 32
G = 16
H = G // 2
NJ = LPA // L


def _bcast(v, lane):
    return jnp.broadcast_to(v[lane], (L,))


def _phase_adaptive_kernel(w2, bias2, fi, vals, ply, out,
                           idx_s, rows_a, rows_b, bidx_v, brow_v,
                           fi_v, vals_v, ply_v, out_v,
                           sem_a, sem_b, sem_bias, sem_out,
                           sem_fi0, sem_fi1, sem_vl0, sem_vl1):
    batch = out.shape[0]
    rows_per_w = batch // NW
    n_groups = rows_per_w // G
    wid = lax.axis_index("s") * 2 + lax.axis_index("c")
    base = pl.multiple_of(wid * rows_per_w, rows_per_w)

    pltpu.sync_copy(ply.at[pl.ds(base, rows_per_w)], ply_v)

    sem_fi = (sem_fi0, sem_fi1)
    sem_vl = (sem_vl0, sem_vl1)

    def fire_fi(g, par):
        gbase = pl.multiple_of(base + g * G, G)
        pltpu.async_copy(fi.at[pl.ds(gbase, G)], fi_v.at[par], sem_fi[par])

    def fire_vals(g, par):
        gbase = pl.multiple_of(base + g * G, G)
        pltpu.async_copy(vals.at[pl.ds(gbase, G)], vals_v.at[par],
                         sem_vl[par])

    def wait_fi(par):
        pltpu.make_async_copy(fi.at[pl.ds(0, G)], fi_v.at[par],
                              sem_fi[par]).wait()

    def wait_vals(par):
        pltpu.make_async_copy(vals.at[pl.ds(0, G)], vals_v.at[par],
                              sem_vl[par]).wait()

    def compute_idx(g, par):
        goff = pl.multiple_of(g * G, G)
        plyg = ply_v[pl.ds(goff, L)]
        bucket = jnp.minimum(lax.div(plyg, jnp.int32(BUCKET_SIZE)),
                             jnp.int32(COUNT - 1))
        bidx_v[par, pl.ds(0, L)] = bucket
        bucket8 = bucket * jnp.int32(8)
        for r in range(G):
            bb8 = _bcast(bucket8, r)
            f0 = fi_v[par, r, pl.ds(0, L)]
            f1 = fi_v[par, r, pl.ds(L, L)]
            k, p = divmod(r, 4)
            idx_s[par, k, pl.ds(p * ACTIVE, L)] = (
                lax.shift_right_logical(f0, 3) * jnp.int32(48) + bb8
                + (f0 & jnp.int32(7)))
            idx_s[par, k, pl.ds(p * ACTIVE + L, L)] = (
                lax.shift_right_logical(f1, 3) * jnp.int32(48) + bb8
                + (f1 & jnp.int32(7)))

    def fire_a(g, par):
        pltpu.async_copy(w2.at[idx_s.at[par, 0]], rows_a.at[pl.ds(0, 128)],
                         sem_a)
        pltpu.async_copy(w2.at[idx_s.at[par, 1]], rows_a.at[pl.ds(128, 128)],
                         sem_a)
        pltpu.async_copy(bias2.at[bidx_v.at[par]], brow_v.at[par], sem_bias)

    def fire_b(g, par):
        pltpu.async_copy(w2.at[idx_s.at[par, 2]], rows_b.at[pl.ds(0, 128)],
                         sem_b)
        pltpu.async_copy(w2.at[idx_s.at[par, 3]], rows_b.at[pl.ds(128, 128)],
                         sem_b)

    def drain(rows_ref, sem, n):
        for _ in range(n):
            pltpu.make_async_copy(w2.at[pl.ds(0, 128)],
                                  rows_ref.at[pl.ds(0, 128)], sem).wait()

    def compute_half(g, par, half, rows_ref):
        def row_body(r, c):
            v0 = vals_v[par, half * H + r, pl.ds(0, L)]
            v1 = vals_v[par, half * H + r, pl.ds(L, L)]
            accs = [brow_v[par, half * H + r, pl.ds(j * L, L)]
                    for j in range(NJ)]
            rbase = r * ACTIVE
            for a in range(ACTIVE):
                w = _bcast(v0 if a < L else v1, a % L)
                for j in range(NJ):
                    accs[j] = accs[j] + w * rows_ref[rbase + a, pl.ds(j * L, L)]
            for j in range(NJ):
                x = jnp.minimum(jnp.maximum(accs[j], 0.0), 1.0)
                out_v[par, half * H + r, pl.ds(j * L, L)] = \
                    x * x * (1023.0 / 1024.0)
            return c

        lax.fori_loop(0, H, row_body, 0)

    fire_fi(0, 0)
    fire_vals(0, 0)
    wait_fi(0)
    wait_vals(0)
    compute_idx(0, 0)
    fire_a(0, 0)
    fire_b(0, 0)
    fire_fi(1, 1)

    def pair_body(gg, carry):
        for p in (0, 1):
            g = gg * 2 + p
            last = g + 1 >= n_groups

            @pl.when(g + 2 < n_groups)
            def _():
                fire_fi(g + 2, p)

            @pl.when(jnp.logical_not(last))
            def _():
                fire_vals(g + 1, 1 - p)
                wait_fi(1 - p)
                compute_idx(g + 1, 1 - p)

            pltpu.make_async_copy(w2.at[pl.ds(0, G)], brow_v.at[p],
                                  sem_bias).wait()
            drain(rows_a, sem_a, 2)

            @pl.when(g >= 2)
            def _():
                pltpu.make_async_copy(out_v.at[p],
                                      out.at[pl.ds(base, G)], sem_out).wait()

            @pl.when(g >= 1)
            def _():
                wait_vals(p)

            compute_half(g, p, 0, rows_a)

            @pl.when(jnp.logical_not(last))
            def _():
                fire_a(g + 1, 1 - p)

            drain(rows_b, sem_b, 2)
            compute_half(g, p, 1, rows_b)

            @pl.when(jnp.logical_not(last))
            def _():
                fire_b(g + 1, 1 - p)

            gbase = pl.multiple_of(base + g * G, G)
            pltpu.async_copy(out_v.at[p], out.at[pl.ds(gbase, G)], sem_out)
        return carry

    lax.fori_loop(0, n_groups // 2, pair_body, 0)

    for p in (0, 1):
        pltpu.make_async_copy(out_v.at[p], out.at[pl.ds(base, G)],
                              sem_out).wait()


def kernel(feature_indices, values, batch_size, in_features, ply, weight, bias):
    del batch_size, in_features
    batch = feature_indices.shape[0]
    fi = feature_indices.astype(jnp.int32)
    vals1 = values
    nf = weight.shape[0]
    w2 = (weight.reshape(nf // 8, 8, COUNT, LPA)
          .transpose(0, 2, 1, 3)
          .reshape(-1, LPA))
    bias2 = bias.reshape(COUNT, LPA)
    ply32 = ply.astype(jnp.int32)
    rows_per_w = batch // NW

    mesh = plsc.VectorSubcoreMesh(core_axis_name="c", subcore_axis_name="s")
    f = functools.partial(
        pl.kernel,
        mesh=mesh,
        out_type=jax.ShapeDtypeStruct((batch, LPA), jnp.float32),
        scratch_types=[
            pltpu.VMEM((2, 4, 128), jnp.int32),
            pltpu.VMEM((H * ACTIVE, LPA), jnp.float32),
            pltpu.VMEM((H * ACTIVE, LPA), jnp.float32),
            pltpu.VMEM((2, L), jnp.int32),
            pltpu.VMEM((2, G, LPA), jnp.float32),
            pltpu.VMEM((2, G, ACTIVE), jnp.int32),
            pltpu.VMEM((2, G, ACTIVE), jnp.float32),
            pltpu.VMEM((rows_per_w,), jnp.int32),
            pltpu.VMEM((2, G, LPA), jnp.float32),
            pltpu.SemaphoreType.DMA,
            pltpu.SemaphoreType.DMA,
            pltpu.SemaphoreType.DMA,
            pltpu.SemaphoreType.DMA,
            pltpu.SemaphoreType.DMA,
            pltpu.SemaphoreType.DMA,
            pltpu.SemaphoreType.DMA,
            pltpu.SemaphoreType.DMA,
        ],
    )(_phase_adaptive_kernel)
    return f(w2, bias2, fi, vals1, ply32)


# speedup vs baseline: 1.0240x; 1.0240x over previous
import functools

import jax
import jax.numpy as jnp
from jax import lax
from jax.experimental import pallas as pl
from jax.experimental.pallas import tpu as pltpu
from jax.experimental.pallas import tpu_sc as plsc

L = 16
LPA = 128
COUNT = 6
BUCKET_SIZE = 5
ACTIVE = 32
NW = 32
G = 16
H = G // 2
NJ = LPA // L


def _bcast(v, lane):
    return jnp.broadcast_to(v[lane], (L,))


def _phase_adaptive_kernel(w2, bias2, fi, vals, ply, out,
                           idx_s, rows_a, rows_b, bidx_v, brow_v,
                           fi_v, vals_v, ply_v, out_v,
                           sem_a, sem_b, sem_bias, sem_out,
                           sem_vl0, sem_vl1):
    batch = out.shape[0]
    rows_per_w = batch // NW
    n_groups = rows_per_w // G
    wid = lax.axis_index("s") * 2 + lax.axis_index("c")
    base = pl.multiple_of(wid * rows_per_w, rows_per_w)

    pltpu.sync_copy(ply.at[pl.ds(base, rows_per_w)], ply_v)
    pltpu.sync_copy(fi.at[pl.ds(0, ACTIVE), pl.ds(base, rows_per_w)], fi_v)

    sem_vl = (sem_vl0, sem_vl1)

    def fire_vals(g, par):
        gbase = pl.multiple_of(base + g * G, G)
        pltpu.async_copy(vals.at[pl.ds(gbase, G)], vals_v.at[par],
                         sem_vl[par])

    def wait_vals(par):
        pltpu.make_async_copy(vals.at[pl.ds(0, G)], vals_v.at[par],
                              sem_vl[par]).wait()

    def compute_idx(g, par):
        goff = pl.multiple_of(g * G, G)
        plyg = ply_v[pl.ds(goff, L)]
        bucket = jnp.minimum(lax.div(plyg, jnp.int32(BUCKET_SIZE)),
                             jnp.int32(COUNT - 1))
        bidx_v[par, pl.ds(0, L)] = bucket
        bucket8 = bucket * jnp.int32(8)
        for a in range(ACTIVE):
            fa = fi_v[a, pl.ds(goff, L)]
            k, p = divmod(a, 8)
            idx_s[par, k, pl.ds(p * L, L)] = (
                lax.shift_right_logical(fa, 3) * jnp.int32(48) + bucket8
                + (fa & jnp.int32(7)))

    def fire_a(g, par):
        pltpu.async_copy(w2.at[idx_s.at[par, 0]], rows_a.at[pl.ds(0, 128)],
                         sem_a)
        pltpu.async_copy(w2.at[idx_s.at[par, 1]], rows_a.at[pl.ds(128, 128)],
                         sem_a)
        pltpu.async_copy(bias2.at[bidx_v.at[par]], brow_v.at[par], sem_bias)

    def fire_b(g, par):
        pltpu.async_copy(w2.at[idx_s.at[par, 2]], rows_b.at[pl.ds(0, 128)],
                         sem_b)
        pltpu.async_copy(w2.at[idx_s.at[par, 3]], rows_b.at[pl.ds(128, 128)],
                         sem_b)

    def drain(rows_ref, sem, n):
        for _ in range(n):
            pltpu.make_async_copy(w2.at[pl.ds(0, 128)],
                                  rows_ref.at[pl.ds(0, 128)], sem).wait()

    def compute_half(g, par, half, rows_ref):
        def row_body(r, c):
            w_vec = vals_v[par, r, pl.ds(half * L, L)]
            if half == 0:
                accs = [brow_v[par, r, pl.ds(j * L, L)] for j in range(NJ)]
            else:
                accs = [out_v[par, r, pl.ds(j * L, L)] for j in range(NJ)]
            for i in range(L):
                w = _bcast(w_vec, i)
                for j in range(NJ):
                    accs[j] = accs[j] + w * rows_ref[i * L + r,
                                                     pl.ds(j * L, L)]
            if half == 0:
                for j in range(NJ):
                    out_v[par, r, pl.ds(j * L, L)] = accs[j]
            else:
                for j in range(NJ):
                    x = jnp.minimum(jnp.maximum(accs[j], 0.0), 1.0)
                    out_v[par, r, pl.ds(j * L, L)] = \
                        x * x * (1023.0 / 1024.0)
            return c

        lax.fori_loop(0, G, row_body, 0)

    fire_vals(0, 0)
    wait_vals(0)
    compute_idx(0, 0)
    fire_a(0, 0)
    fire_b(0, 0)

    def pair_body(gg, carry):
        for p in (0, 1):
            g = gg * 2 + p
            last = g + 1 >= n_groups

            @pl.when(jnp.logical_not(last))
            def _():
                fire_vals(g + 1, 1 - p)
                compute_idx(g + 1, 1 - p)

            pltpu.make_async_copy(w2.at[pl.ds(0, G)], brow_v.at[p],
                                  sem_bias).wait()
            drain(rows_a, sem_a, 2)

            @pl.when(g >= 2)
            def _():
                pltpu.make_async_copy(out_v.at[p],
                                      out.at[pl.ds(base, G)], sem_out).wait()

            @pl.when(g >= 1)
            def _():
                wait_vals(p)

            compute_half(g, p, 0, rows_a)

            @pl.when(jnp.logical_not(last))
            def _():
                fire_a(g + 1, 1 - p)

            drain(rows_b, sem_b, 2)
            compute_half(g, p, 1, rows_b)

            @pl.when(jnp.logical_not(last))
            def _():
                fire_b(g + 1, 1 - p)

            gbase = pl.multiple_of(base + g * G, G)
            pltpu.async_copy(out_v.at[p], out.at[pl.ds(gbase, G)], sem_out)
        return carry

    lax.fori_loop(0, n_groups // 2, pair_body, 0)

    for p in (0, 1):
        pltpu.make_async_copy(out_v.at[p], out.at[pl.ds(base, G)],
                              sem_out).wait()


def kernel(feature_indices, values, batch_size, in_features, ply, weight, bias):
    del batch_size, in_features
    batch = feature_indices.shape[0]
    fi = feature_indices.astype(jnp.int32).T
    vals1 = values
    nf = weight.shape[0]
    w2 = (weight.reshape(nf // 8, 8, COUNT, LPA)
          .transpose(0, 2, 1, 3)
          .reshape(-1, LPA))
    bias2 = bias.reshape(COUNT, LPA)
    ply32 = ply.astype(jnp.int32)
    rows_per_w = batch // NW

    mesh = plsc.VectorSubcoreMesh(core_axis_name="c", subcore_axis_name="s")
    f = functools.partial(
        pl.kernel,
        mesh=mesh,
        out_type=jax.ShapeDtypeStruct((batch, LPA), jnp.float32),
        scratch_types=[
            pltpu.VMEM((2, 4, 128), jnp.int32),
            pltpu.VMEM((H * ACTIVE, LPA), jnp.float32),
            pltpu.VMEM((H * ACTIVE, LPA), jnp.float32),
            pltpu.VMEM((2, L), jnp.int32),
            pltpu.VMEM((2, G, LPA), jnp.float32),
            pltpu.VMEM((ACTIVE, rows_per_w), jnp.int32),
            pltpu.VMEM((2, G, ACTIVE), jnp.float32),
            pltpu.VMEM((rows_per_w,), jnp.int32),
            pltpu.VMEM((2, G, LPA), jnp.float32),
            pltpu.SemaphoreType.DMA,
            pltpu.SemaphoreType.DMA,
            pltpu.SemaphoreType.DMA,
            pltpu.SemaphoreType.DMA,
            pltpu.SemaphoreType.DMA,
            pltpu.SemaphoreType.DMA,
        ],
    )(_phase_adaptive_kernel)
    return f(w2, bias2, fi, vals1, ply32)
